# baseline (speedup 1.0000x reference)
import functools

import jax
import jax.numpy as jnp
from jax import lax
from jax.experimental import pallas as pl
from jax.experimental.pallas import tpu as pltpu
from jax.experimental.pallas import tpu_sc as plsc

_N = 16384
_C = 1000
_NC = 2
_NS = 16
_NW = _NC * _NS
_LANES = 16
_TILE_W = 128
_CHUNK = 128

_N_SC = 8192
_RPW = _N_SC // _NW
_N_TC = _N - _N_SC
_TC_BLK = 512


def _sc_body(vt_hbm, idx_hbm, out_hbm, idx_v, lines_v, out_v, sem):
    wid = lax.axis_index("s") * _NC + lax.axis_index("c")
    base = wid * _RPW

    pltpu.sync_copy(idx_hbm.at[pl.ds(base, _RPW)], idx_v)

    copies = []
    for c in range(_RPW // _CHUNK):
        win = pl.multiple_of(base + c * _CHUNK, _TILE_W)
        copies.append(pltpu.async_copy(
            vt_hbm.at[idx_v.at[pl.ds(c * _CHUNK, _CHUNK)],
                      pl.ds(win, _TILE_W)],
            lines_v.at[pl.ds(c * _CHUNK, _CHUNK), :],
            sem,
        ))
    for cp in copies:
        cp.wait()

    lane = lax.iota(jnp.int32, _LANES)
    for k in range(_RPW // _LANES):
        o = lane + k * _LANES
        col = jnp.bitwise_and(o, _TILE_W - 1)
        out_v[pl.ds(k * _LANES, _LANES)] = plsc.load_gather(lines_v, [o, col])

    pltpu.sync_copy(out_v, out_hbm.at[pl.ds(base, _RPW)])


def _sc_gather(vt, idx):
    mesh = plsc.VectorSubcoreMesh(core_axis_name="c", subcore_axis_name="s")
    run = functools.partial(
        pl.kernel,
        out_type=jax.ShapeDtypeStruct((_N_SC,), jnp.float32),
        mesh=mesh,
        compiler_params=pltpu.CompilerParams(needs_layout_passes=False),
        scratch_types=[
            pltpu.VMEM((_RPW,), jnp.int32),
            pltpu.VMEM((_RPW, _TILE_W), jnp.float32),
            pltpu.VMEM((_RPW,), jnp.float32),
            pltpu.SemaphoreType.DMA,
        ],
    )(_sc_body)
    return run(vt, idx)


def _tc_body(idx_ref, t_ref, out_ref):
    rows = lax.broadcasted_iota(jnp.int32, (_C, _TC_BLK), 0)
    sel = rows == idx_ref[0]
    out_ref[0, 0, :] = jnp.sum(jnp.where(sel, t_ref[...], 0.0), axis=0)


def _tc_reduce(vt, idx_all):
    nblk = _N_TC // _TC_BLK
    blk0 = _N_SC // _TC_BLK
    out = pl.pallas_call(
        _tc_body,
        grid=(nblk,),
        in_specs=[
            pl.BlockSpec((1, 1, _TC_BLK), lambda c: (blk0 + c, 0, 0)),
            pl.BlockSpec((_C, _TC_BLK), lambda c: (0, blk0 + c)),
        ],
        out_specs=pl.BlockSpec((1, 1, _TC_BLK), lambda c: (c, 0, 0)),
        out_shape=jax.ShapeDtypeStruct((nblk, 1, _TC_BLK), jnp.float32),
    )(idx_all.reshape(_N // _TC_BLK, 1, _TC_BLK), vt)
    return out.reshape(_N_TC)


@jax.jit
def kernel(value, index):
    vt = value.T
    idx = index.reshape(_N).astype(jnp.int32)
    tc_out = _tc_reduce(vt, idx)
    sc_out = _sc_gather(vt, idx)
    return jnp.concatenate([sc_out, tc_out]).reshape(_N, 1)


# speedup vs baseline: 1.3087x; 1.3087x over previous
import functools

import jax
import jax.numpy as jnp
from jax import lax
from jax.experimental import pallas as pl
from jax.experimental.pallas import tpu as pltpu
from jax.experimental.pallas import tpu_sc as plsc

_N = 16384
_C = 1000
_NC = 2
_NS = 16
_NW = _NC * _NS
_RPW = _N // _NW
_LANES = 16
_TILE_W = 128
_CHUNK = 128


def _sc_body(vt_hbm, idx_hbm, out_hbm, idx_v, lines_v, out_v, sem):
    wid = lax.axis_index("s") * _NC + lax.axis_index("c")
    base = wid * _RPW

    pltpu.sync_copy(idx_hbm.at[pl.ds(base, _RPW)], idx_v)

    copies = []
    for c in range(_RPW // _CHUNK):
        win = pl.multiple_of(base + c * _CHUNK, _TILE_W)
        copies.append(pltpu.async_copy(
            vt_hbm.at[idx_v.at[pl.ds(c * _CHUNK, _CHUNK)],
                      pl.ds(win, _TILE_W)],
            lines_v.at[pl.ds(c * _CHUNK, _CHUNK), :],
            sem,
        ))
    for cp in copies:
        cp.wait()

    lane = lax.iota(jnp.int32, _LANES)
    for k in range(_RPW // _LANES):
        o = lane + k * _LANES
        col = jnp.bitwise_and(o, _TILE_W - 1)
        out_v[pl.ds(k * _LANES, _LANES)] = plsc.load_gather(lines_v, [o, col])

    pltpu.sync_copy(out_v, out_hbm.at[pl.ds(base, _RPW)])


@jax.jit
def kernel(value, index):
    mesh = plsc.VectorSubcoreMesh(core_axis_name="c", subcore_axis_name="s")
    run = functools.partial(
        pl.kernel,
        out_type=jax.ShapeDtypeStruct((_N,), jnp.float32),
        mesh=mesh,
        compiler_params=pltpu.CompilerParams(needs_layout_passes=False),
        scratch_types=[
            pltpu.VMEM((_RPW,), jnp.int32),
            pltpu.VMEM((_RPW, _TILE_W), jnp.float32),
            pltpu.VMEM((_RPW,), jnp.float32),
            pltpu.SemaphoreType.DMA,
        ],
    )(_sc_body)
    flat = run(value.T, index.reshape(_N).astype(jnp.int32))
    return flat.reshape(_N, 1)
